# baseline (speedup 1.0000x reference)
import functools

import jax
import jax.numpy as jnp
from jax import lax
from jax.experimental import pallas as pl
from jax.experimental.pallas import tpu as pltpu
from jax.experimental.pallas import tpu_sc as plsc

B, H, W = 4, 384, 384
N = B * H * W
NLAB = 32
BINS = 48
NCHUNK = 32
CH = N // NCHUNK
NV = CH // 16
UNROLL = 8
ROWP = 256
BH_B = 96


def _sc_kernel(lab_hbm, dy_hbm, dx_hbm, dm_hbm,
               lab_v, dy_v, dx_v, dm_v, acc, acc_b, mdy_v, mdx_v, comb_v,
               shared):
    c = lax.axis_index("c")
    s = lax.axis_index("s")
    img_in_core = s // 8
    chunk = s % 8
    base = (c * 2 + img_in_core) * (H * W) + chunk * CH

    pltpu.sync_copy(lab_hbm.at[pl.ds(base, CH)], lab_v)
    pltpu.sync_copy(dy_hbm.at[pl.ds(base, CH)], dy_v)
    pltpu.sync_copy(dx_hbm.at[pl.ds(base, CH)], dx_v)

    zeros16 = jnp.zeros((16,), jnp.float32)
    for j in range(3 * BINS // 16):
        acc[pl.ds(j * 16, 16)] = zeros16
        acc_b[pl.ds(j * 16, 16)] = zeros16

    ones16 = jnp.ones((16,), jnp.float32)

    def scat_body(i, carry):
        off = i * (UNROLL * 16)
        for j in range(UNROLL):
            sl = pl.ds(off + j * 16, 16)
            idx = lab_v[sl]
            tgt = acc if j % 2 == 0 else acc_b
            plsc.addupdate_scatter(tgt, [idx], ones16)
            plsc.addupdate_scatter(tgt, [idx + BINS], dy_v[sl])
            plsc.addupdate_scatter(tgt, [idx + 2 * BINS], dx_v[sl])
        return carry

    lax.fori_loop(0, NV // UNROLL, scat_body, 0)
    for j in range(3 * BINS // 16):
        sl = pl.ds(j * 16, 16)
        acc[sl] = acc[sl] + acc_b[sl]

    pltpu.sync_copy(acc, shared.at[s])
    plsc.subcore_barrier()
    pltpu.sync_copy(shared.at[pl.ds(img_in_core * 8, 8)], comb_v)

    lane = lax.iota(jnp.int32, 16)
    for j in range(BINS // 16):
        sl = pl.ds(j * 16, 16)
        sly = pl.ds(BINS + j * 16, 16)
        slx = pl.ds(2 * BINS + j * 16, 16)
        cnt = comb_v[0, sl]
        sdy = comb_v[0, sly]
        sdx = comb_v[0, slx]
        for k in range(1, 8):
            cnt = cnt + comb_v[k, sl]
            sdy = sdy + comb_v[k, sly]
            sdx = sdx + comb_v[k, slx]
        inv = 1.0 / jnp.maximum(cnt, 1.0)
        mdy = sdy * inv
        mdx = sdx * inv
        if j == 0:
            mdy = jnp.where(lane == 0, 0.0, mdy)
            mdx = jnp.where(lane == 0, 0.0, mdx)
        mdy_v[sl] = mdy
        mdx_v[sl] = mdx

    def gath_body(i, carry):
        off = i * (UNROLL * 16)
        for j in range(UNROLL):
            sl = pl.ds(off + j * 16, 16)
            idx = lab_v[sl]
            gdy = plsc.load_gather(mdy_v, [idx])
            gdx = plsc.load_gather(mdx_v, [idx])
            d1 = gdy - dy_v[sl]
            d2 = gdx - dx_v[sl]
            dm_v[sl] = d1 * d1 + d2 * d2
        return carry

    lax.fori_loop(0, NV // UNROLL, gath_body, 0)

    pltpu.sync_copy(dm_v, dm_hbm.at[pl.ds(base, CH)])


_sc_call = pl.kernel(
    _sc_kernel,
    out_type=jax.ShapeDtypeStruct((N,), jnp.float32),
    mesh=plsc.VectorSubcoreMesh(core_axis_name="c", subcore_axis_name="s"),
    scratch_types=[
        pltpu.VMEM((CH,), jnp.int32),
        pltpu.VMEM((CH,), jnp.float32),
        pltpu.VMEM((CH,), jnp.float32),
        pltpu.VMEM((CH,), jnp.float32),
        pltpu.VMEM((ROWP,), jnp.float32),
        pltpu.VMEM((ROWP,), jnp.float32),
        pltpu.VMEM((BINS,), jnp.float32),
        pltpu.VMEM((BINS,), jnp.float32),
        pltpu.VMEM((8, ROWP), jnp.float32),
        pltpu.VMEM_SHARED((16, ROWP), jnp.float32),
    ],
    compiler_params=pltpu.CompilerParams(needs_layout_passes=False),
)


def _dense_kernel(edm_t_ref, edm_p_ref, c0_ref, c1_ref, c2_ref, c3_ref,
                  cat_t_ref, out_ref):
    edm_l = jnp.square(edm_t_ref[0] - edm_p_ref[0])

    ct = cat_t_ref[0]
    c0 = c0_ref[0]
    c1 = c1_ref[0]
    c2 = c2_ref[0]
    c3 = c3_ref[0]
    ssum = c0 + c1 + c2 + c3
    pt = jnp.where(ct == 1, c1, c0)
    pt = jnp.where(ct == 2, c2, pt)
    pt = jnp.where(ct == 3, c3, pt)
    p = jnp.clip(pt / ssum, 1e-7, 1.0 - 1e-7)
    w = jnp.where(ct >= 2, 5.0, 1.0)
    cat_l = -jnp.log(p) * w

    out_ref[0] = edm_l * (1.0 / 3.0) + cat_l * (1.0 / 3.0)


def _combine_kernel(partial_ref, dm_ref, out_ref):
    out_ref[0] = partial_ref[0] + dm_ref[0] * (1.0 / 6.0)


@jax.jit
def kernel(edm_true, edm_pred, dy_pred, dx_pred, cat_pred, cat_true, labels):
    edm_t = edm_true.reshape(B, H, W)
    edm_p = edm_pred.reshape(B, H, W)
    ct = cat_true.reshape(B, H, W)
    cats = [cat_pred[..., c] for c in range(4)]

    dm = _sc_call(labels.reshape(N), dy_pred.reshape(N), dx_pred.reshape(N))
    dm = dm.reshape(B, H, W)

    nt = H // BH_B
    partial = pl.pallas_call(
        _dense_kernel,
        grid=(B, nt),
        in_specs=[pl.BlockSpec((1, BH_B, W), lambda b, t: (b, t, 0))
                  for _ in range(7)],
        out_specs=pl.BlockSpec((1, BH_B, W), lambda b, t: (b, t, 0)),
        out_shape=jax.ShapeDtypeStruct((B, H, W), jnp.float32),
        compiler_params=pltpu.CompilerParams(
            allow_input_fusion=[True] * 7),
    )(edm_t, edm_p, *cats, ct)

    loss = pl.pallas_call(
        _combine_kernel,
        grid=(B,),
        in_specs=[
            pl.BlockSpec((1, H, W), lambda b: (b, 0, 0)),
            pl.BlockSpec((1, H, W), lambda b: (b, 0, 0)),
        ],
        out_specs=pl.BlockSpec((1, H, W), lambda b: (b, 0, 0)),
        out_shape=jax.ShapeDtypeStruct((B, H, W), jnp.float32),
        compiler_params=pltpu.CompilerParams(
            allow_input_fusion=[True, True]),
    )(partial, dm)
    return loss


# speedup vs baseline: 1.0122x; 1.0122x over previous
import functools

import jax
import jax.numpy as jnp
from jax import lax
from jax.experimental import pallas as pl
from jax.experimental.pallas import tpu as pltpu
from jax.experimental.pallas import tpu_sc as plsc

B, H, W = 4, 384, 384
N = B * H * W
NLAB = 32
BINS = 48
NCHUNK = 32
CH = N // NCHUNK
NV = CH // 16
UNROLL = 8
ROWP = 256
BH_B = 96


def _sc_kernel(lab_hbm, dy_hbm, dx_hbm, dm_hbm,
               lab_v, dy_v, dx_v, dm_v, acc, acc_b, mdy_v, mdx_v, comb_v,
               shared, sem):
    c = lax.axis_index("c")
    s = lax.axis_index("s")
    img_in_core = s // 8
    chunk = s % 8
    base = (c * 2 + img_in_core) * (H * W) + chunk * CH

    cp1 = pltpu.async_copy(lab_hbm.at[pl.ds(base, CH)], lab_v, sem)
    cp2 = pltpu.async_copy(dy_hbm.at[pl.ds(base, CH)], dy_v, sem)
    cp3 = pltpu.async_copy(dx_hbm.at[pl.ds(base, CH)], dx_v, sem)
    cp1.wait()
    cp2.wait()
    cp3.wait()

    zeros16 = jnp.zeros((16,), jnp.float32)
    for j in range(3 * BINS // 16):
        acc[pl.ds(j * 16, 16)] = zeros16
        acc_b[pl.ds(j * 16, 16)] = zeros16

    ones16 = jnp.ones((16,), jnp.float32)

    def scat_body(i, carry):
        off = i * (UNROLL * 16)
        for j in range(UNROLL):
            sl = pl.ds(off + j * 16, 16)
            idx = lab_v[sl]
            tgt = acc if j % 2 == 0 else acc_b
            plsc.addupdate_scatter(tgt, [idx], ones16)
            plsc.addupdate_scatter(tgt, [idx + BINS], dy_v[sl])
            plsc.addupdate_scatter(tgt, [idx + 2 * BINS], dx_v[sl])
        return carry

    lax.fori_loop(0, NV // UNROLL, scat_body, 0)
    for j in range(3 * BINS // 16):
        sl = pl.ds(j * 16, 16)
        acc[sl] = acc[sl] + acc_b[sl]

    pltpu.sync_copy(acc, shared.at[s])
    plsc.subcore_barrier()
    pltpu.sync_copy(shared.at[pl.ds(img_in_core * 8, 8)], comb_v)

    lane = lax.iota(jnp.int32, 16)
    for j in range(BINS // 16):
        sl = pl.ds(j * 16, 16)
        sly = pl.ds(BINS + j * 16, 16)
        slx = pl.ds(2 * BINS + j * 16, 16)
        cnt = comb_v[0, sl]
        sdy = comb_v[0, sly]
        sdx = comb_v[0, slx]
        for k in range(1, 8):
            cnt = cnt + comb_v[k, sl]
            sdy = sdy + comb_v[k, sly]
            sdx = sdx + comb_v[k, slx]
        inv = 1.0 / jnp.maximum(cnt, 1.0)
        mdy = sdy * inv
        mdx = sdx * inv
        if j == 0:
            mdy = jnp.where(lane == 0, 0.0, mdy)
            mdx = jnp.where(lane == 0, 0.0, mdx)
        mdy_v[sl] = mdy
        mdx_v[sl] = mdx

    def gath_body(i, carry):
        off = i * (UNROLL * 16)
        for j in range(UNROLL):
            sl = pl.ds(off + j * 16, 16)
            idx = lab_v[sl]
            gdy = plsc.load_gather(mdy_v, [idx])
            gdx = plsc.load_gather(mdx_v, [idx])
            d1 = gdy - dy_v[sl]
            d2 = gdx - dx_v[sl]
            dm_v[sl] = d1 * d1 + d2 * d2
        return carry

    lax.fori_loop(0, NV // UNROLL, gath_body, 0)

    pltpu.sync_copy(dm_v, dm_hbm.at[pl.ds(base, CH)])


_sc_call = pl.kernel(
    _sc_kernel,
    out_type=jax.ShapeDtypeStruct((N,), jnp.float32),
    mesh=plsc.VectorSubcoreMesh(core_axis_name="c", subcore_axis_name="s"),
    scratch_types=[
        pltpu.VMEM((CH,), jnp.int32),
        pltpu.VMEM((CH,), jnp.float32),
        pltpu.VMEM((CH,), jnp.float32),
        pltpu.VMEM((CH,), jnp.float32),
        pltpu.VMEM((ROWP,), jnp.float32),
        pltpu.VMEM((ROWP,), jnp.float32),
        pltpu.VMEM((BINS,), jnp.float32),
        pltpu.VMEM((BINS,), jnp.float32),
        pltpu.VMEM((8, ROWP), jnp.float32),
        pltpu.VMEM_SHARED((16, ROWP), jnp.float32),
        pltpu.SemaphoreType.DMA,
    ],
    compiler_params=pltpu.CompilerParams(needs_layout_passes=False),
)


def _dense_kernel(edm_t_ref, edm_p_ref, c0_ref, c1_ref, c2_ref, c3_ref,
                  cat_t_ref, out_ref):
    edm_l = jnp.square(edm_t_ref[0] - edm_p_ref[0])

    ct = cat_t_ref[0]
    c0 = c0_ref[0]
    c1 = c1_ref[0]
    c2 = c2_ref[0]
    c3 = c3_ref[0]
    ssum = c0 + c1 + c2 + c3
    pt = jnp.where(ct == 1, c1, c0)
    pt = jnp.where(ct == 2, c2, pt)
    pt = jnp.where(ct == 3, c3, pt)
    p = jnp.clip(pt / ssum, 1e-7, 1.0 - 1e-7)
    w = jnp.where(ct >= 2, 5.0, 1.0)
    cat_l = -jnp.log(p) * w

    out_ref[0] = edm_l * (1.0 / 3.0) + cat_l * (1.0 / 3.0)


def _combine_kernel(partial_ref, dm_ref, out_ref):
    out_ref[0] = partial_ref[0] + dm_ref[0] * (1.0 / 6.0)


@jax.jit
def kernel(edm_true, edm_pred, dy_pred, dx_pred, cat_pred, cat_true, labels):
    edm_t = edm_true.reshape(B, H, W)
    edm_p = edm_pred.reshape(B, H, W)
    ct = cat_true.reshape(B, H, W)
    cats = [cat_pred[..., c] for c in range(4)]

    dm = _sc_call(labels.reshape(N), dy_pred.reshape(N), dx_pred.reshape(N))
    dm = dm.reshape(B, H, W)

    nt = H // BH_B
    partial = pl.pallas_call(
        _dense_kernel,
        grid=(B, nt),
        in_specs=[pl.BlockSpec((1, BH_B, W), lambda b, t: (b, t, 0))
                  for _ in range(7)],
        out_specs=pl.BlockSpec((1, BH_B, W), lambda b, t: (b, t, 0)),
        out_shape=jax.ShapeDtypeStruct((B, H, W), jnp.float32),
        compiler_params=pltpu.CompilerParams(
            allow_input_fusion=[True] * 7),
    )(edm_t, edm_p, *cats, ct)

    loss = pl.pallas_call(
        _combine_kernel,
        grid=(B,),
        in_specs=[
            pl.BlockSpec((1, H, W), lambda b: (b, 0, 0)),
            pl.BlockSpec((1, H, W), lambda b: (b, 0, 0)),
        ],
        out_specs=pl.BlockSpec((1, H, W), lambda b: (b, 0, 0)),
        out_shape=jax.ShapeDtypeStruct((B, H, W), jnp.float32),
        compiler_params=pltpu.CompilerParams(
            allow_input_fusion=[True, True]),
    )(partial, dm)
    return loss


# speedup vs baseline: 1.0129x; 1.0007x over previous
import functools

import jax
import jax.numpy as jnp
from jax import lax
from jax.experimental import pallas as pl
from jax.experimental.pallas import tpu as pltpu
from jax.experimental.pallas import tpu_sc as plsc

B, H, W = 4, 384, 384
N = B * H * W
NLAB = 32
BINS = 48
NCHUNK = 32
CH = N // NCHUNK
NV = CH // 16
UNROLL = 8
ROWP = 256
BH_B = 96


def _sc_kernel(lab_hbm, dy_hbm, dx_hbm, dm_hbm,
               lab_v, dy_v, dx_v, dm_v, acc, acc_b, mdy_v, mdx_v, comb_v,
               shared, sem):
    c = lax.axis_index("c")
    s = lax.axis_index("s")
    img_in_core = s // 8
    chunk = s % 8
    base = (c * 2 + img_in_core) * (H * W) + chunk * CH

    cp1 = pltpu.async_copy(lab_hbm.at[pl.ds(base, CH)], lab_v, sem)
    cp2 = pltpu.async_copy(dy_hbm.at[pl.ds(base, CH)], dy_v, sem)
    cp3 = pltpu.async_copy(dx_hbm.at[pl.ds(base, CH)], dx_v, sem)
    cp1.wait()
    cp2.wait()
    cp3.wait()

    zeros16 = jnp.zeros((16,), jnp.float32)
    for j in range(3 * BINS // 16):
        acc[pl.ds(j * 16, 16)] = zeros16
        acc_b[pl.ds(j * 16, 16)] = zeros16

    ones16 = jnp.ones((16,), jnp.float32)

    def scat_body(i, carry):
        off = i * (UNROLL * 16)
        for j in range(UNROLL):
            sl = pl.ds(off + j * 16, 16)
            idx = lab_v[sl]
            tgt = acc if j % 2 == 0 else acc_b
            plsc.addupdate_scatter(tgt, [idx], ones16)
            plsc.addupdate_scatter(tgt, [idx + BINS], dy_v[sl])
            plsc.addupdate_scatter(tgt, [idx + 2 * BINS], dx_v[sl])
        return carry

    lax.fori_loop(0, NV // UNROLL, scat_body, 0)
    for j in range(3 * BINS // 16):
        sl = pl.ds(j * 16, 16)
        acc[sl] = acc[sl] + acc_b[sl]

    pltpu.sync_copy(acc, shared.at[s])
    plsc.subcore_barrier()
    pltpu.sync_copy(shared.at[pl.ds(img_in_core * 8, 8)], comb_v)

    lane = lax.iota(jnp.int32, 16)
    for j in range(BINS // 16):
        sl = pl.ds(j * 16, 16)
        sly = pl.ds(BINS + j * 16, 16)
        slx = pl.ds(2 * BINS + j * 16, 16)
        cnt = comb_v[0, sl]
        sdy = comb_v[0, sly]
        sdx = comb_v[0, slx]
        for k in range(1, 8):
            cnt = cnt + comb_v[k, sl]
            sdy = sdy + comb_v[k, sly]
            sdx = sdx + comb_v[k, slx]
        inv = 1.0 / jnp.maximum(cnt, 1.0)
        mdy = sdy * inv
        mdx = sdx * inv
        if j == 0:
            mdy = jnp.where(lane == 0, 0.0, mdy)
            mdx = jnp.where(lane == 0, 0.0, mdx)
        mdy_v[sl] = mdy
        mdx_v[sl] = mdx

    GUN = 16

    def gath_body(i, carry):
        off = i * (GUN * 16)
        for j in range(GUN):
            sl = pl.ds(off + j * 16, 16)
            idx = lab_v[sl]
            gdy = plsc.load_gather(mdy_v, [idx])
            gdx = plsc.load_gather(mdx_v, [idx])
            d1 = gdy - dy_v[sl]
            d2 = gdx - dx_v[sl]
            dm_v[sl] = d1 * d1 + d2 * d2
        return carry

    lax.fori_loop(0, NV // GUN, gath_body, 0)

    pltpu.sync_copy(dm_v, dm_hbm.at[pl.ds(base, CH)])


_sc_call = pl.kernel(
    _sc_kernel,
    out_type=jax.ShapeDtypeStruct((N,), jnp.float32),
    mesh=plsc.VectorSubcoreMesh(core_axis_name="c", subcore_axis_name="s"),
    scratch_types=[
        pltpu.VMEM((CH,), jnp.int32),
        pltpu.VMEM((CH,), jnp.float32),
        pltpu.VMEM((CH,), jnp.float32),
        pltpu.VMEM((CH,), jnp.float32),
        pltpu.VMEM((ROWP,), jnp.float32),
        pltpu.VMEM((ROWP,), jnp.float32),
        pltpu.VMEM((BINS,), jnp.float32),
        pltpu.VMEM((BINS,), jnp.float32),
        pltpu.VMEM((8, ROWP), jnp.float32),
        pltpu.VMEM_SHARED((16, ROWP), jnp.float32),
        pltpu.SemaphoreType.DMA,
    ],
    compiler_params=pltpu.CompilerParams(needs_layout_passes=False),
)


def _dense_kernel(edm_t_ref, edm_p_ref, c0_ref, c1_ref, c2_ref, c3_ref,
                  cat_t_ref, out_ref):
    edm_l = jnp.square(edm_t_ref[0] - edm_p_ref[0])

    ct = cat_t_ref[0]
    c0 = c0_ref[0]
    c1 = c1_ref[0]
    c2 = c2_ref[0]
    c3 = c3_ref[0]
    ssum = c0 + c1 + c2 + c3
    pt = jnp.where(ct == 1, c1, c0)
    pt = jnp.where(ct == 2, c2, pt)
    pt = jnp.where(ct == 3, c3, pt)
    p = jnp.clip(pt / ssum, 1e-7, 1.0 - 1e-7)
    w = jnp.where(ct >= 2, 5.0, 1.0)
    cat_l = -jnp.log(p) * w

    out_ref[0] = edm_l * (1.0 / 3.0) + cat_l * (1.0 / 3.0)


def _combine_kernel(partial_ref, dm_ref, out_ref):
    out_ref[0] = partial_ref[0] + dm_ref[0] * (1.0 / 6.0)


@jax.jit
def kernel(edm_true, edm_pred, dy_pred, dx_pred, cat_pred, cat_true, labels):
    edm_t = edm_true.reshape(B, H, W)
    edm_p = edm_pred.reshape(B, H, W)
    ct = cat_true.reshape(B, H, W)
    cats = [cat_pred[..., c] for c in range(4)]

    dm = _sc_call(labels.reshape(N), dy_pred.reshape(N), dx_pred.reshape(N))
    dm = dm.reshape(B, H, W)

    nt = H // BH_B
    partial = pl.pallas_call(
        _dense_kernel,
        grid=(B, nt),
        in_specs=[pl.BlockSpec((1, BH_B, W), lambda b, t: (b, t, 0))
                  for _ in range(7)],
        out_specs=pl.BlockSpec((1, BH_B, W), lambda b, t: (b, t, 0)),
        out_shape=jax.ShapeDtypeStruct((B, H, W), jnp.float32),
        compiler_params=pltpu.CompilerParams(
            allow_input_fusion=[True] * 7),
    )(edm_t, edm_p, *cats, ct)

    loss = pl.pallas_call(
        _combine_kernel,
        grid=(B,),
        in_specs=[
            pl.BlockSpec((1, H, W), lambda b: (b, 0, 0)),
            pl.BlockSpec((1, H, W), lambda b: (b, 0, 0)),
        ],
        out_specs=pl.BlockSpec((1, H, W), lambda b: (b, 0, 0)),
        out_shape=jax.ShapeDtypeStruct((B, H, W), jnp.float32),
        compiler_params=pltpu.CompilerParams(
            allow_input_fusion=[True, True]),
    )(partial, dm)
    return loss
